# baseline (speedup 1.0000x reference)
import jax
import jax.numpy as jnp
from jax.experimental import pallas as pl
from jax.experimental.pallas import tpu as pltpu

N = 2048
NFEAT = 512
NHID = 64
DHID1 = 64
C = 3

_DEC_BLK = 64
_NBLK = N // _DEC_BLK


def _dsn_apply(h, W1, b1, W2, b2, W3, b3):
    h = jax.nn.relu(jnp.dot(h, W1, preferred_element_type=jnp.float32) + b1)
    h = jax.nn.relu(jnp.dot(h, W2, preferred_element_type=jnp.float32) + b2)
    return jnp.dot(h, W3, preferred_element_type=jnp.float32) + b3


def _fused_kernel(x_ref, ap_ref, aa_ref, an_ref, sim_ref,
                  w1_ref, w2_ref,
                  dW1_ref, db1_ref, dW2_ref, db2_ref, dW3_ref, db3_ref,
                  aW1_ref, ab1_ref, aW2_ref, ab2_ref, aW3_ref, ab3_ref,
                  dec_ref, out_ref,
                  abuf, x_scr, emb_scr, sem, xsem):
    i = pl.program_id(0)

    adj_refs = [ap_ref, aa_ref, an_ref]

    @pl.when(i == 0)
    def _gcn_and_combine():
        def copy_k(k):
            v, e = divmod(k, 3)
            return pltpu.make_async_copy(
                adj_refs[e].at[v], abuf.at[k % 2], sem.at[k % 2])

        xcopy = pltpu.make_async_copy(x_ref, x_scr, xsem)
        xcopy.start()
        copy_k(0).start()
        copy_k(1).start()
        xcopy.wait()
        P = jnp.dot(x_scr[...], w1_ref[:, :5 * NHID],
                    preferred_element_type=jnp.float32)
        G, D = [], []
        for k in range(9):
            if k == 5:
                P = jnp.dot(x_scr[...], w1_ref[:, 5 * NHID:],
                            preferred_element_type=jnp.float32)
            Pk = P[:, (k - 5 if k >= 5 else k) * NHID:][:, :NHID]
            copy_k(k).wait()
            A = abuf[k % 2]
            H = jax.nn.relu(jnp.dot(A, Pk,
                                    preferred_element_type=jnp.float32))
            HW2 = jnp.dot(H, w2_ref[k], preferred_element_type=jnp.float32)
            G.append(jnp.dot(A, HW2, preferred_element_type=jnp.float32))
            if k + 2 < 9:
                copy_k(k + 2).start()
            if k % 3 == 2:
                v = k // 3
                E = jnp.concatenate(G, axis=1)
                G = []
                D.append(_dsn_apply(E, dW1_ref[v], db1_ref[v:v + 1, :],
                                    dW2_ref[v], db2_ref[v:v + 1, :],
                                    dW3_ref[v], db3_ref[v:v + 1, :]))
        sub = jnp.concatenate([sim_ref[0:1, 0:1] * D[1],
                               sim_ref[0:1, 1:2] * D[2]], axis=1)
        agg = _dsn_apply(sub, aW1_ref[...], ab1_ref[...], aW2_ref[...],
                         ab2_ref[...], aW3_ref[...], ab3_ref[...])
        emb_scr[...] = jnp.concatenate([D[0], agg], axis=1)

    eblk = emb_scr[pl.ds(i * _DEC_BLK, _DEC_BLK), :]
    t = jnp.dot(eblk, dec_ref[...], preferred_element_type=jnp.float32)
    z = jax.lax.dot_general(t, emb_scr[...], (((1,), (1,)), ((), ())),
                            preferred_element_type=jnp.float32)
    out_ref[...] = jax.nn.sigmoid(z)


def kernel(x, adjs_pos, adjs_add, adjs_neg, attW, enc_W1, enc_W2,
           dsn_W1, dsn_b1, dsn_W2, dsn_b2, dsn_W3, dsn_b3,
           agg_W1, agg_b1, agg_W2, agg_b2, agg_W3, agg_b3, dec_W):
    w1all = enc_W1.reshape(9, NFEAT, NHID).transpose(1, 0, 2).reshape(
        NFEAT, 9 * NHID)
    w2all = enc_W2.reshape(9, NHID, NHID)
    sim = jax.nn.softmax(attW, axis=0).reshape(1, C - 1)

    full = lambda s: pl.BlockSpec(s, lambda i: tuple(0 for _ in s))
    hbm = pl.BlockSpec(memory_space=pltpu.MemorySpace.HBM)
    return pl.pallas_call(
        _fused_kernel,
        grid=(_NBLK,),
        in_specs=[
            hbm, hbm, hbm, hbm,
            full((1, C - 1)),
            full((NFEAT, 9 * NHID)), full((9, NHID, NHID)),
            full((C, 3 * NHID, DHID1)), full((C, DHID1)),
            full((C, DHID1, 2 * DHID1)), full((C, 2 * DHID1)),
            full((C, 2 * DHID1, DHID1)), full((C, DHID1)),
            full((2 * DHID1, 2 * DHID1)), full((1, 2 * DHID1)),
            full((2 * DHID1, 4 * DHID1)), full((1, 4 * DHID1)),
            full((4 * DHID1, DHID1)), full((1, DHID1)),
            full((2 * DHID1, 2 * DHID1)),
        ],
        out_specs=pl.BlockSpec((_DEC_BLK, N), lambda i: (i, 0)),
        out_shape=jax.ShapeDtypeStruct((N, N), jnp.float32),
        compiler_params=pltpu.CompilerParams(vmem_limit_bytes=66_000_000),
        scratch_shapes=[
            pltpu.VMEM((2, N, N), jnp.float32),
            pltpu.VMEM((N, NFEAT), jnp.float32),
            pltpu.VMEM((N, 2 * DHID1), jnp.float32),
            pltpu.SemaphoreType.DMA((2,)),
            pltpu.SemaphoreType.DMA,
        ],
    )(x, adjs_pos, adjs_add, adjs_neg, sim, w1all, w2all,
      dsn_W1, dsn_b1, dsn_W2, dsn_b2, dsn_W3, dsn_b3,
      agg_W1, agg_b1.reshape(1, -1), agg_W2, agg_b2.reshape(1, -1),
      agg_W3, agg_b3.reshape(1, -1), dec_W)


# speedup vs baseline: 1.1327x; 1.1327x over previous
import jax
import jax.numpy as jnp
from jax.experimental import pallas as pl
from jax.experimental.pallas import tpu as pltpu

N = 2048
NFEAT = 512
NHID = 64
DHID1 = 64
C = 3

_DEC_BLK = 256
_NBLK = N // _DEC_BLK


def _dsn_apply(h, W1, b1, W2, b2, W3, b3):
    h = jax.nn.relu(jnp.dot(h, W1, preferred_element_type=jnp.float32) + b1)
    h = jax.nn.relu(jnp.dot(h, W2, preferred_element_type=jnp.float32) + b2)
    return jnp.dot(h, W3, preferred_element_type=jnp.float32) + b3


def _fused_kernel(x_ref, ap_ref, aa_ref, an_ref, sim_ref,
                  w1_ref, w2_ref,
                  dW1_ref, db1_ref, dW2_ref, db2_ref, dW3_ref, db3_ref,
                  aW1_ref, ab1_ref, aW2_ref, ab2_ref, aW3_ref, ab3_ref,
                  dec_ref, out_ref,
                  abuf, emb_scr, sem):
    i = pl.program_id(0)

    adj_refs = [ap_ref, aa_ref, an_ref]

    @pl.when(i == 0)
    def _gcn_and_combine():
        def copy_u(h):
            k, half = divmod(h, 2)
            v, e = divmod(k, 3)
            return pltpu.make_async_copy(
                adj_refs[e].at[v, pl.ds(half * (N // 2), N // 2), :],
                abuf.at[h % 3], sem.at[h % 3])

        copy_u(0).start()
        copy_u(1).start()
        G = []
        Pp = None
        for k in range(9):
            if k % 2 == 0:
                hi = min(k + 2, 9) * NHID
                Pp = jnp.dot(x_ref[...], w1_ref[:, k * NHID:hi],
                             preferred_element_type=jnp.float32)
            Pk = Pp[:, (k % 2) * NHID:(k % 2 + 1) * NHID]
            if k < 8:
                copy_u(2 * k + 2).start()
            copy_u(2 * k).wait()
            A_top = abuf[(2 * k) % 3]
            H_top = jax.nn.relu(jnp.dot(A_top, Pk,
                                        preferred_element_type=jnp.float32))
            HW2_top = jnp.dot(H_top, w2_ref[k],
                              preferred_element_type=jnp.float32)
            copy_u(2 * k + 1).wait()
            A_bot = abuf[(2 * k + 1) % 3]
            H_bot = jax.nn.relu(jnp.dot(A_bot, Pk,
                                        preferred_element_type=jnp.float32))
            HW2_bot = jnp.dot(H_bot, w2_ref[k],
                              preferred_element_type=jnp.float32)
            HW2 = jnp.concatenate([HW2_top, HW2_bot], axis=0)
            G_top = jnp.dot(A_top, HW2, preferred_element_type=jnp.float32)
            if k < 8:
                copy_u(2 * k + 3).start()
            G_bot = jnp.dot(A_bot, HW2, preferred_element_type=jnp.float32)
            G.append(jnp.concatenate([G_top, G_bot], axis=0))

        D = []
        for v in range(C):
            E = jnp.concatenate(G[3 * v:3 * v + 3], axis=1)
            D.append(_dsn_apply(E, dW1_ref[v], db1_ref[v:v + 1, :],
                                dW2_ref[v], db2_ref[v:v + 1, :],
                                dW3_ref[v], db3_ref[v:v + 1, :]))
        sub = jnp.concatenate([sim_ref[0:1, 0:1] * D[1],
                               sim_ref[0:1, 1:2] * D[2]], axis=1)
        agg = _dsn_apply(sub, aW1_ref[...], ab1_ref[...], aW2_ref[...],
                         ab2_ref[...], aW3_ref[...], ab3_ref[...])
        emb_scr[...] = jnp.concatenate([D[0], agg], axis=1)

    eblk = emb_scr[pl.ds(i * _DEC_BLK, _DEC_BLK), :]
    t = jnp.dot(eblk, dec_ref[...], preferred_element_type=jnp.float32)
    z = jax.lax.dot_general(t, emb_scr[...], (((1,), (1,)), ((), ())),
                            preferred_element_type=jnp.float32)
    out_ref[...] = jax.nn.sigmoid(z)


def kernel(x, adjs_pos, adjs_add, adjs_neg, attW, enc_W1, enc_W2,
           dsn_W1, dsn_b1, dsn_W2, dsn_b2, dsn_W3, dsn_b3,
           agg_W1, agg_b1, agg_W2, agg_b2, agg_W3, agg_b3, dec_W):
    w1all = enc_W1.reshape(9, NFEAT, NHID).transpose(1, 0, 2).reshape(
        NFEAT, 9 * NHID)
    w2all = enc_W2.reshape(9, NHID, NHID)
    sim = jax.nn.softmax(attW, axis=0).reshape(1, C - 1)

    full = lambda s: pl.BlockSpec(s, lambda i: tuple(0 for _ in s))
    hbm = pl.BlockSpec(memory_space=pltpu.MemorySpace.HBM)
    return pl.pallas_call(
        _fused_kernel,
        grid=(_NBLK,),
        in_specs=[
            full((N, NFEAT)), hbm, hbm, hbm,
            full((1, C - 1)),
            full((NFEAT, 9 * NHID)), full((9, NHID, NHID)),
            full((C, 3 * NHID, DHID1)), full((C, DHID1)),
            full((C, DHID1, 2 * DHID1)), full((C, 2 * DHID1)),
            full((C, 2 * DHID1, DHID1)), full((C, DHID1)),
            full((2 * DHID1, 2 * DHID1)), full((1, 2 * DHID1)),
            full((2 * DHID1, 4 * DHID1)), full((1, 4 * DHID1)),
            full((4 * DHID1, DHID1)), full((1, DHID1)),
            full((2 * DHID1, 2 * DHID1)),
        ],
        out_specs=pl.BlockSpec((_DEC_BLK, N), lambda i: (i, 0)),
        out_shape=jax.ShapeDtypeStruct((N, N), jnp.float32),
        scratch_shapes=[
            pltpu.VMEM((3, N // 2, N), jnp.float32),
            pltpu.VMEM((N, 2 * DHID1), jnp.float32),
            pltpu.SemaphoreType.DMA((3,)),
        ],
    )(x, adjs_pos, adjs_add, adjs_neg, sim, w1all, w2all,
      dsn_W1, dsn_b1, dsn_W2, dsn_b2, dsn_W3, dsn_b3,
      agg_W1, agg_b1.reshape(1, -1), agg_W2, agg_b2.reshape(1, -1),
      agg_W3, agg_b3.reshape(1, -1), dec_W)


# speedup vs baseline: 1.2731x; 1.1239x over previous
import jax
import jax.numpy as jnp
from jax.experimental import pallas as pl
from jax.experimental.pallas import tpu as pltpu

N = 2048
NFEAT = 512
NHID = 64
DHID1 = 64
C = 3

_DEC_BLK = 256
_NBLK = N // _DEC_BLK


def _dsn_apply(h, W1, b1, W2, b2, W3, b3):
    h = jax.nn.relu(jnp.dot(h, W1, preferred_element_type=jnp.float32) + b1)
    h = jax.nn.relu(jnp.dot(h, W2, preferred_element_type=jnp.float32) + b2)
    return jnp.dot(h, W3, preferred_element_type=jnp.float32) + b3


def _fused_kernel(x_ref, ap_ref, aa_ref, an_ref, sim_ref,
                  w1_ref, w2_ref,
                  dW1_ref, db1_ref, dW2_ref, db2_ref, dW3_ref, db3_ref,
                  aW1_ref, ab1_ref, aW2_ref, ab2_ref, aW3_ref, ab3_ref,
                  dec_ref, out_ref,
                  abuf, emb_scr, sem):
    i = pl.program_id(0)

    adj_refs = [ap_ref, aa_ref, an_ref]

    @pl.when(i == 0)
    def _gcn_and_combine():
        def copy_u(h):
            k, half = divmod(h, 2)
            v, e = divmod(k, 3)
            return pltpu.make_async_copy(
                adj_refs[e].at[v, pl.ds(half * (N // 2), N // 2), :],
                abuf.at[h % 3], sem.at[h % 3])

        copy_u(0).start()
        copy_u(1).start()
        G = []
        for k in range(9):
            if k < 8:
                copy_u(2 * k + 2).start()
            Pk = jnp.dot(x_ref[...], w1_ref[k],
                         preferred_element_type=jnp.float32)
            copy_u(2 * k).wait()
            A_top = abuf[(2 * k) % 3]
            H_top = jax.nn.relu(jnp.dot(A_top, Pk,
                                        preferred_element_type=jnp.float32))
            copy_u(2 * k + 1).wait()
            A_bot = abuf[(2 * k + 1) % 3]
            H_bot = jax.nn.relu(jnp.dot(A_bot, Pk,
                                        preferred_element_type=jnp.float32))
            HW2 = jnp.dot(jnp.concatenate([H_top, H_bot], axis=0), w2_ref[k],
                          preferred_element_type=jnp.float32)
            G_top = jnp.dot(A_top, HW2, preferred_element_type=jnp.float32)
            if k < 8:
                copy_u(2 * k + 3).start()
            G_bot = jnp.dot(A_bot, HW2, preferred_element_type=jnp.float32)
            G.append(jnp.concatenate([G_top, G_bot], axis=0))

        D = []
        for v in range(C):
            E = jnp.concatenate(G[3 * v:3 * v + 3], axis=1)
            D.append(_dsn_apply(E, dW1_ref[v], db1_ref[v:v + 1, :],
                                dW2_ref[v], db2_ref[v:v + 1, :],
                                dW3_ref[v], db3_ref[v:v + 1, :]))
        sub = jnp.concatenate([sim_ref[0:1, 0:1] * D[1],
                               sim_ref[0:1, 1:2] * D[2]], axis=1)
        agg = _dsn_apply(sub, aW1_ref[...], ab1_ref[...], aW2_ref[...],
                         ab2_ref[...], aW3_ref[...], ab3_ref[...])
        emb_scr[...] = jnp.concatenate([D[0], agg], axis=1)

    eblk = emb_scr[pl.ds(i * _DEC_BLK, _DEC_BLK), :]
    t = jnp.dot(eblk, dec_ref[...], preferred_element_type=jnp.float32)
    z = jax.lax.dot_general(t, emb_scr[...], (((1,), (1,)), ((), ())),
                            preferred_element_type=jnp.float32)
    out_ref[...] = jax.nn.sigmoid(z)


def kernel(x, adjs_pos, adjs_add, adjs_neg, attW, enc_W1, enc_W2,
           dsn_W1, dsn_b1, dsn_W2, dsn_b2, dsn_W3, dsn_b3,
           agg_W1, agg_b1, agg_W2, agg_b2, agg_W3, agg_b3, dec_W):
    w1all = enc_W1.reshape(9, NFEAT, NHID)
    w2all = enc_W2.reshape(9, NHID, NHID)
    sim = jax.nn.softmax(attW, axis=0).reshape(1, C - 1)

    full = lambda s: pl.BlockSpec(s, lambda i: tuple(0 for _ in s))
    hbm = pl.BlockSpec(memory_space=pltpu.MemorySpace.HBM)
    return pl.pallas_call(
        _fused_kernel,
        grid=(_NBLK,),
        in_specs=[
            full((N, NFEAT)), hbm, hbm, hbm,
            full((1, C - 1)),
            full((9, NFEAT, NHID)), full((9, NHID, NHID)),
            full((C, 3 * NHID, DHID1)), full((C, DHID1)),
            full((C, DHID1, 2 * DHID1)), full((C, 2 * DHID1)),
            full((C, 2 * DHID1, DHID1)), full((C, DHID1)),
            full((2 * DHID1, 2 * DHID1)), full((1, 2 * DHID1)),
            full((2 * DHID1, 4 * DHID1)), full((1, 4 * DHID1)),
            full((4 * DHID1, DHID1)), full((1, DHID1)),
            full((2 * DHID1, 2 * DHID1)),
        ],
        out_specs=pl.BlockSpec((_DEC_BLK, N), lambda i: (i, 0)),
        out_shape=jax.ShapeDtypeStruct((N, N), jnp.float32),
        scratch_shapes=[
            pltpu.VMEM((3, N // 2, N), jnp.float32),
            pltpu.VMEM((N, 2 * DHID1), jnp.float32),
            pltpu.SemaphoreType.DMA((3,)),
        ],
    )(x, adjs_pos, adjs_add, adjs_neg, sim, w1all, w2all,
      dsn_W1, dsn_b1, dsn_W2, dsn_b2, dsn_W3, dsn_b3,
      agg_W1, agg_b1.reshape(1, -1), agg_W2, agg_b2.reshape(1, -1),
      agg_W3, agg_b3.reshape(1, -1), dec_W)
